# baseline (speedup 1.0000x reference)
import jax
import jax.numpy as jnp
import numpy as np
from jax import lax
from jax.experimental import pallas as pl
from jax.experimental.pallas import tpu as pltpu
from jax.experimental.pallas import tpu_sc as plsc


def _selected_rows(C, bank_size):
    key = jax.random.key(42)
    fg_rows = jax.random.permutation(key, bank_size - 1)[: C - 1] + 1
    bg_rows = jnp.zeros((1,), dtype=fg_rows.dtype)
    return jnp.concatenate([bg_rows, fg_rows])


def _dense_body(rowmap_ref, pos_ref, dense_ref, dense_out_ref):
    i = pl.program_id(0)
    row = rowmap_ref[i]
    enc = pos_ref[row, :]
    dense_out_ref[...] = dense_ref[...] + enc[None, None, :]


def _dense_call(rowmap, pos2, dense3, G, HW, D, bank_size):
    grid_spec = pltpu.PrefetchScalarGridSpec(
        num_scalar_prefetch=1,
        grid=(G,),
        in_specs=[
            pl.BlockSpec((bank_size, D), lambda i, rm: (0, 0)),
            pl.BlockSpec((1, HW, D), lambda i, rm: (i, 0, 0)),
        ],
        out_specs=[
            pl.BlockSpec((1, HW, D), lambda i, rm: (i, 0, 0)),
        ],
    )
    (dense_out,) = pl.pallas_call(
        _dense_body,
        grid_spec=grid_spec,
        out_shape=[jax.ShapeDtypeStruct((G, HW, D), jnp.float32)],
        compiler_params=pltpu.CompilerParams(
            dimension_semantics=("arbitrary",),
        ),
    )(rowmap, pos2, dense3)
    return dense_out


def _make_sc_sparse(G, N, D, C, bank_size):
    NC, NS = 2, 16
    NW = NC * NS
    slabs_per_w = G // NW
    mesh = plsc.VectorSubcoreMesh(core_axis_name="c", subcore_axis_name="s")

    def body(rows_hbm, pos_hbm, sparse_hbm, out_hbm, idx_v, enc_v, x_v, sem):
        wid = lax.axis_index("s") * NC + lax.axis_index("c")
        pltpu.sync_copy(rows_hbm, idx_v)
        pltpu.async_copy(pos_hbm.at[idx_v], enc_v, sem).wait()
        for k in range(slabs_per_w):
            slab = wid * slabs_per_w + k
            c = lax.rem(slab, C)
            pltpu.sync_copy(sparse_hbm.at[slab], x_v)
            for j in range(D // 16):
                e = enc_v[c, pl.ds(j * 16, 16)]
                for r in range(N):
                    x_v[r, pl.ds(j * 16, 16)] = x_v[r, pl.ds(j * 16, 16)] + e
            pltpu.sync_copy(x_v, out_hbm.at[slab])

    return pl.kernel(
        body,
        out_type=jax.ShapeDtypeStruct((G, N, D), jnp.float32),
        mesh=mesh,
        scratch_types=[
            pltpu.VMEM((C,), jnp.int32),
            pltpu.VMEM((C, D), jnp.float32),
            pltpu.VMEM((N, D), jnp.float32),
            pltpu.SemaphoreType.DMA,
        ],
    )


def kernel(dense_embeddings, sparse_embeddings, pos_embedding):
    B, M, C, N, D = sparse_embeddings.shape
    _, _, _, _, H, W = dense_embeddings.shape
    bank_size = pos_embedding.shape[2]
    G = B * M * C
    HW = H * W

    try:
        with jax.ensure_compile_time_eval():
            rows_np = np.asarray(_selected_rows(C, bank_size)).astype(np.int32)
        rows = jnp.asarray(rows_np)
        rowmap = jnp.asarray(np.tile(rows_np, B * M))
    except Exception:
        rows = _selected_rows(C, bank_size).astype(jnp.int32)
        rowmap = jnp.tile(rows, B * M)

    dense3 = dense_embeddings.transpose(0, 1, 2, 4, 5, 3).reshape(G, HW, D)
    sparse3 = sparse_embeddings.reshape(G, N, D)
    pos2 = pos_embedding.reshape(bank_size, D)

    dense_out = _dense_call(rowmap, pos2, dense3, G, HW, D, bank_size)
    sparse_out = _make_sc_sparse(G, N, D, C, bank_size)(rows, pos2, sparse3)

    dense_out = dense_out.reshape(B, M, C, H, W, D).transpose(0, 1, 2, 5, 3, 4)
    return (dense_out, sparse_out.reshape(B, M, C, N, D))


# speedup vs baseline: 1.0013x; 1.0013x over previous
import jax
import jax.numpy as jnp
import numpy as np
from jax import lax
from jax.experimental import pallas as pl
from jax.experimental.pallas import tpu as pltpu
from jax.experimental.pallas import tpu_sc as plsc


def _selected_rows(C, bank_size):
    key = jax.random.key(42)
    fg_rows = jax.random.permutation(key, bank_size - 1)[: C - 1] + 1
    bg_rows = jnp.zeros((1,), dtype=fg_rows.dtype)
    return jnp.concatenate([bg_rows, fg_rows])


def _dense_body(rowmap_ref, pos_ref, dense_ref, dense_out_ref):
    i = pl.program_id(0)
    row = rowmap_ref[i]
    enc = pos_ref[row, :]
    dense_out_ref[...] = dense_ref[...] + enc[None, None, :]


def _dense_call(rowmap, pos2, dense3, G, HW, D, bank_size):
    grid_spec = pltpu.PrefetchScalarGridSpec(
        num_scalar_prefetch=1,
        grid=(G,),
        in_specs=[
            pl.BlockSpec((bank_size, D), lambda i, rm: (0, 0)),
            pl.BlockSpec((1, HW, D), lambda i, rm: (i, 0, 0)),
        ],
        out_specs=[
            pl.BlockSpec((1, HW, D), lambda i, rm: (i, 0, 0)),
        ],
    )
    (dense_out,) = pl.pallas_call(
        _dense_body,
        grid_spec=grid_spec,
        out_shape=[jax.ShapeDtypeStruct((G, HW, D), jnp.float32)],
        compiler_params=pltpu.CompilerParams(
            dimension_semantics=("arbitrary",),
        ),
    )(rowmap, pos2, dense3)
    return dense_out


def _make_sc_sparse(G, N, D, C, bank_size):
    NC, NS = 2, 16
    NW = NC * NS
    slabs_per_w = G // NW
    mesh = plsc.VectorSubcoreMesh(core_axis_name="c", subcore_axis_name="s")

    def body(rows_hbm, pos_hbm, sparse_hbm, out_hbm, idx_v, enc_v, x_v, sem):
        wid = lax.axis_index("s") * NC + lax.axis_index("c")
        pltpu.sync_copy(rows_hbm, idx_v)
        pltpu.async_copy(pos_hbm.at[idx_v], enc_v, sem).wait()
        for k in range(slabs_per_w):
            slab = wid * slabs_per_w + k
            c = lax.rem(slab, C)
            pltpu.sync_copy(sparse_hbm.at[slab], x_v)

            def _row(r, _):
                for j in range(D // 16):
                    e = enc_v[c, pl.ds(j * 16, 16)]
                    x_v[r, pl.ds(j * 16, 16)] = x_v[r, pl.ds(j * 16, 16)] + e
                return 0

            lax.fori_loop(0, N, _row, 0)
            pltpu.sync_copy(x_v, out_hbm.at[slab])

    return pl.kernel(
        body,
        out_type=jax.ShapeDtypeStruct((G, N, D), jnp.float32),
        mesh=mesh,
        scratch_types=[
            pltpu.VMEM((C,), jnp.int32),
            pltpu.VMEM((C, D), jnp.float32),
            pltpu.VMEM((N, D), jnp.float32),
            pltpu.SemaphoreType.DMA,
        ],
    )


def kernel(dense_embeddings, sparse_embeddings, pos_embedding):
    B, M, C, N, D = sparse_embeddings.shape
    _, _, _, _, H, W = dense_embeddings.shape
    bank_size = pos_embedding.shape[2]
    G = B * M * C
    HW = H * W

    try:
        with jax.ensure_compile_time_eval():
            rows_np = np.asarray(_selected_rows(C, bank_size)).astype(np.int32)
        rows = jnp.asarray(rows_np)
        rowmap = jnp.asarray(np.tile(rows_np, B * M))
    except Exception:
        rows = _selected_rows(C, bank_size).astype(jnp.int32)
        rowmap = jnp.tile(rows, B * M)

    dense3 = dense_embeddings.transpose(0, 1, 2, 4, 5, 3).reshape(G, HW, D)
    sparse3 = sparse_embeddings.reshape(G, N, D)
    pos2 = pos_embedding.reshape(bank_size, D)

    dense_out = _dense_call(rowmap, pos2, dense3, G, HW, D, bank_size)
    sparse_out = _make_sc_sparse(G, N, D, C, bank_size)(rows, pos2, sparse3)

    dense_out = dense_out.reshape(B, M, C, H, W, D).transpose(0, 1, 2, 5, 3, 4)
    return (dense_out, sparse_out.reshape(B, M, C, N, D))


# speedup vs baseline: 1.1131x; 1.1116x over previous
import jax
import jax.numpy as jnp
import numpy as np
from jax.experimental import pallas as pl
from jax.experimental.pallas import tpu as pltpu


def _selected_rows(C, bank_size):
    key = jax.random.key(42)
    fg_rows = jax.random.permutation(key, bank_size - 1)[: C - 1] + 1
    bg_rows = jnp.zeros((1,), dtype=fg_rows.dtype)
    return jnp.concatenate([bg_rows, fg_rows])


def _encode_body(rowmap_ref, pos_ref, dense_ref, sparse_ref,
                 dense_out_ref, sparse_out_ref):
    i = pl.program_id(0)
    for k in range(dense_ref.shape[0]):
        row = rowmap_ref[i * dense_ref.shape[0] + k]
        enc = pos_ref[row, :]
        dense_out_ref[k, :, :] = dense_ref[k, :, :] + enc[None, :]
        sparse_out_ref[k, :, :] = sparse_ref[k, :, :] + enc[None, :]


def kernel(dense_embeddings, sparse_embeddings, pos_embedding):
    B, M, C, N, D = sparse_embeddings.shape
    _, _, _, _, H, W = dense_embeddings.shape
    bank_size = pos_embedding.shape[2]
    G = B * M * C
    HW = H * W
    SLABS = 2

    try:
        with jax.ensure_compile_time_eval():
            rows_np = np.asarray(_selected_rows(C, bank_size)).astype(np.int32)
        rowmap = jnp.asarray(np.tile(rows_np, B * M))
    except Exception:
        rows = _selected_rows(C, bank_size).astype(jnp.int32)
        rowmap = jnp.tile(rows, B * M)

    dense3 = dense_embeddings.transpose(0, 1, 2, 4, 5, 3).reshape(G, HW, D)
    sparse3 = sparse_embeddings.reshape(G, N, D)
    pos2 = pos_embedding.reshape(bank_size, D)

    grid_spec = pltpu.PrefetchScalarGridSpec(
        num_scalar_prefetch=1,
        grid=(G // SLABS,),
        in_specs=[
            pl.BlockSpec((bank_size, D), lambda i, rm: (0, 0)),
            pl.BlockSpec((SLABS, HW, D), lambda i, rm: (i, 0, 0)),
            pl.BlockSpec((SLABS, N, D), lambda i, rm: (i, 0, 0)),
        ],
        out_specs=[
            pl.BlockSpec((SLABS, HW, D), lambda i, rm: (i, 0, 0)),
            pl.BlockSpec((SLABS, N, D), lambda i, rm: (i, 0, 0)),
        ],
    )

    dense_out, sparse_out = pl.pallas_call(
        _encode_body,
        grid_spec=grid_spec,
        out_shape=[
            jax.ShapeDtypeStruct((G, HW, D), jnp.float32),
            jax.ShapeDtypeStruct((G, N, D), jnp.float32),
        ],
        compiler_params=pltpu.CompilerParams(
            dimension_semantics=("arbitrary",),
        ),
    )(rowmap, pos2, dense3, sparse3)

    dense_out = dense_out.reshape(B, M, C, H, W, D).transpose(0, 1, 2, 5, 3, 4)
    return (dense_out, sparse_out.reshape(B, M, C, N, D))


# speedup vs baseline: 1.1135x; 1.0003x over previous
import jax
import jax.numpy as jnp
import numpy as np
from jax.experimental import pallas as pl
from jax.experimental.pallas import tpu as pltpu


def _selected_rows(C, bank_size):
    key = jax.random.key(42)
    fg_rows = jax.random.permutation(key, bank_size - 1)[: C - 1] + 1
    bg_rows = jnp.zeros((1,), dtype=fg_rows.dtype)
    return jnp.concatenate([bg_rows, fg_rows])


def _encode_body(rowmap_ref, pos_ref, dense_ref, sparse_ref,
                 dense_out_ref, sparse_out_ref):
    i = pl.program_id(0)
    for k in range(dense_ref.shape[0]):
        row = rowmap_ref[i * dense_ref.shape[0] + k]
        enc = pos_ref[row, :]
        dense_out_ref[k, :, :] = dense_ref[k, :, :] + enc[None, :]
        sparse_out_ref[k, :, :] = sparse_ref[k, :, :] + enc[None, :]


def kernel(dense_embeddings, sparse_embeddings, pos_embedding):
    B, M, C, N, D = sparse_embeddings.shape
    _, _, _, _, H, W = dense_embeddings.shape
    bank_size = pos_embedding.shape[2]
    G = B * M * C
    HW = H * W
    SLABS = 2

    try:
        with jax.ensure_compile_time_eval():
            rows_np = np.asarray(_selected_rows(C, bank_size)).astype(np.int32)
        rowmap = jnp.asarray(np.tile(rows_np, B * M))
    except Exception:
        rows = _selected_rows(C, bank_size).astype(jnp.int32)
        rowmap = jnp.tile(rows, B * M)

    dense3 = dense_embeddings.transpose(0, 1, 2, 4, 5, 3).reshape(G, HW, D)
    sparse3 = sparse_embeddings.reshape(G, N, D)
    pos2 = pos_embedding.reshape(bank_size, D)

    grid_spec = pltpu.PrefetchScalarGridSpec(
        num_scalar_prefetch=1,
        grid=(G // SLABS,),
        in_specs=[
            pl.BlockSpec((bank_size, D), lambda i, rm: (0, 0)),
            pl.BlockSpec((SLABS, HW, D), lambda i, rm: (i, 0, 0)),
            pl.BlockSpec((SLABS, N, D), lambda i, rm: (i, 0, 0)),
        ],
        out_specs=[
            pl.BlockSpec((SLABS, HW, D), lambda i, rm: (i, 0, 0)),
            pl.BlockSpec((SLABS, N, D), lambda i, rm: (i, 0, 0)),
        ],
    )

    dense_out, sparse_out = pl.pallas_call(
        _encode_body,
        grid_spec=grid_spec,
        out_shape=[
            jax.ShapeDtypeStruct((G, HW, D), jnp.float32),
            jax.ShapeDtypeStruct((G, N, D), jnp.float32),
        ],
        compiler_params=pltpu.CompilerParams(
            dimension_semantics=("parallel",),
        ),
    )(rowmap, pos2, dense3, sparse3)

    dense_out = dense_out.reshape(B, M, C, H, W, D).transpose(0, 1, 2, 5, 3, 4)
    return (dense_out, sparse_out.reshape(B, M, C, N, D))
